# baseline (speedup 1.0000x reference)
import jax
import jax.numpy as jnp
from jax.experimental import pallas as pl
from jax.experimental.pallas import tpu as pltpu
from jax.experimental.pallas import tpu_sc as plsc

_B, _N, _C = 4, 20000, 80
_R, _L = 160, 128
_RT = 16
_NP = _R * _L
_SCORE_TH = 0.05
_IOU_TH = 0.5
_PRE = 2000
_POST = 100
_IMG = 512.0
_OFF = 2.0 * _IMG

_CR, _CL = 16, 128
_CP = 2560
_TRASH = 2400
_CHUNK = _NP // 16


def _decode_kernel(sc_ref, bx_ref, rg_ref, s_ref, lb_ref, xy_ref):
    def cls_body(c, carry):
        best, besti = carry
        v = sc_ref[0, c]
        gt = v > best
        return jnp.where(gt, v, best), jnp.where(gt, c, besti)

    best0 = jnp.full((_RT, _L), -jnp.inf, jnp.float32)
    besti0 = jnp.zeros((_RT, _L), jnp.int32)
    best, labels = jax.lax.fori_loop(0, _C, cls_body, (best0, besti0))

    t = pl.program_id(1)
    rowi = jax.lax.broadcasted_iota(jnp.int32, (_RT, _L), 0) + t * _RT
    coli = jax.lax.broadcasted_iota(jnp.int32, (_RT, _L), 1)
    gidx = rowi * _L + coli
    pad = gidx >= _N
    valid = best > _SCORE_TH
    s_ref[0] = jnp.where(pad, -2.0, jnp.where(valid, best, -1.0))
    lb_ref[0] = labels

    bx1 = bx_ref[0, 0]
    by1 = bx_ref[0, 1]
    bx2 = bx_ref[0, 2]
    by2 = bx_ref[0, 3]
    dx = rg_ref[0, 0] * 0.1
    dy = rg_ref[0, 1] * 0.1
    dw = rg_ref[0, 2] * 0.2
    dh = rg_ref[0, 3] * 0.2
    w = bx2 - bx1
    h = by2 - by1
    cx = bx1 + 0.5 * w
    cy = by1 + 0.5 * h
    pcx = cx + dx * w
    pcy = cy + dy * h
    pw = w * jnp.exp(dw)
    ph = h * jnp.exp(dh)
    xy_ref[0, 0] = jnp.clip(pcx - 0.5 * pw, 0.0, _IMG)
    xy_ref[0, 1] = jnp.clip(pcy - 0.5 * ph, 0.0, _IMG)
    xy_ref[0, 2] = jnp.clip(pcx + 0.5 * pw, 0.0, _IMG)
    xy_ref[0, 3] = jnp.clip(pcy + 0.5 * ph, 0.0, _IMG)


def _rsum(x):
    return jnp.sum(jnp.sum(x, axis=1, keepdims=True), axis=2, keepdims=True)


def _rmax(x):
    return jnp.max(jnp.max(x, axis=1, keepdims=True), axis=2, keepdims=True)


def _rmin(x):
    return jnp.min(jnp.min(x, axis=1, keepdims=True), axis=2, keepdims=True)


def _topk_kernel(s_ref, in2k_ref, prt_ref):
    s = s_ref[...]
    rowi = jax.lax.broadcasted_iota(jnp.int32, (1, _R, _L), 1)
    coli = jax.lax.broadcasted_iota(jnp.int32, (1, _R, _L), 2)
    gidx = rowi * _L + coli

    bits = jax.lax.bitcast_convert_type(s, jnp.int32)
    key = jnp.where(bits >= 0, bits, -1 - (bits & jnp.int32(0x7FFFFFFF)))

    def bs_body(_, lohi):
        lo, hi = lohi
        mid = (lo >> 1) + (hi >> 1) + (lo & hi & 1)
        big = _rsum((key > mid).astype(jnp.int32)) >= _PRE
        return jnp.where(big, mid, lo), jnp.where(big, hi, mid)

    lo0 = jnp.full((_B, 1, 1), -(2**31), jnp.int32)
    hi0 = jnp.full((_B, 1, 1), 2**31 - 1, jnp.int32)
    _, tau = jax.lax.fori_loop(0, 32, bs_body, (lo0, hi0))
    n1 = _rsum((key > tau).astype(jnp.int32))
    extra = _PRE - n1
    eq = key == tau

    def bs2_body(_, lohi):
        lo, hi = lohi
        mid = (lo + hi) >> 1
        geq = _rsum((eq & (gidx < mid)).astype(jnp.int32)) >= extra
        return jnp.where(geq, lo, mid), jnp.where(geq, mid, hi)

    _, mstar = jax.lax.fori_loop(
        0, 16, bs2_body,
        (jnp.zeros((_B, 1, 1), jnp.int32), jnp.full((_B, 1, 1), _NP, jnp.int32)),
    )
    in2k = ((key > tau) | (eq & (gidx < mstar))).astype(jnp.int32)
    in2k_ref[...] = in2k
    for t in range(16):
        prt_ref[:, t:t + 1, :] = jnp.sum(
            in2k[:, t * 10:(t + 1) * 10, :], axis=1, keepdims=True
        )


def _compact_kernel(mask_hbm, prt_hbm, s_hbm, lb_hbm, x1_hbm, y1_hbm,
                    x2_hbm, y2_hbm,
                    so_hbm, go_hbm, lo_hbm, xo1_hbm, yo1_hbm, xo2_hbm,
                    yo2_hbm,
                    prt_v, mask_v, pos_v, gidx_v, ffld_v, ifld_v, sem):
    c = jax.lax.axis_index("c")
    t = jax.lax.axis_index("s")
    lanes = jax.lax.iota(jnp.int32, 16)

    def gbody(k, _):
        gidx_v[pl.ds(k * 16, 16)] = t * _CHUNK + k * 16 + lanes
        return 0

    jax.lax.fori_loop(0, _CHUNK // 16, gbody, 0)

    for b_loc in range(2):
        b = c * 2 + b_loc
        pltpu.sync_copy(prt_hbm.at[pl.ds(b * 16 * _L, 16 * _L)], prt_v)

        def base_body(k, acc):
            v = prt_v[pl.ds(k * 16, 16)]
            take = (k >> 3) < t
            return acc + jnp.where(take, jnp.sum(v), 0)

        base = jax.lax.fori_loop(0, 16 * _L // 16, base_body, jnp.int32(0))

        off_in = b * _NP + t * _CHUNK
        pltpu.sync_copy(mask_hbm.at[pl.ds(off_in, _CHUNK)], mask_v)

        def pos_body(k, run):
            m = mask_v[pl.ds(k * 16, 16)]
            incl = plsc.cumsum(m)
            pos = (incl - m) + run
            tgt = jnp.where(m > 0, b * _CP + pos, b * _CP + _TRASH)
            pos_v[pl.ds(k * 16, 16)] = tgt
            return run + jnp.sum(m)

        jax.lax.fori_loop(0, _CHUNK // 16, pos_body, base)

        for src, dst in ((s_hbm, so_hbm), (x1_hbm, xo1_hbm),
                         (y1_hbm, yo1_hbm), (x2_hbm, xo2_hbm),
                         (y2_hbm, yo2_hbm)):
            pltpu.sync_copy(src.at[pl.ds(off_in, _CHUNK)], ffld_v)
            pltpu.async_copy(ffld_v, dst.at[pos_v], sem).wait()
        pltpu.sync_copy(lb_hbm.at[pl.ds(off_in, _CHUNK)], ifld_v)
        pltpu.async_copy(ifld_v, lo_hbm.at[pos_v], sem).wait()
        pltpu.async_copy(gidx_v, go_hbm.at[pos_v], sem).wait()


def _nms_kernel(s_ref, g_ref, lb_ref, x1_ref, y1_ref, x2_ref, y2_ref,
                bo_ref, so_ref, lo_ref):
    rowi = jax.lax.broadcasted_iota(jnp.int32, (1, _CR, _CL), 1)
    coli = jax.lax.broadcasted_iota(jnp.int32, (1, _CR, _CL), 2)
    cpos = rowi * _CL + coli
    ok = cpos < _PRE
    s = jnp.where(ok, s_ref[...], -2.0)
    gidx = jnp.where(ok, g_ref[...], _NP)
    labels = jnp.where(ok, lb_ref[...], 0)
    x1 = jnp.where(ok, x1_ref[...], 0.0)
    y1 = jnp.where(ok, y1_ref[...], 0.0)
    x2 = jnp.where(ok, x2_ref[...], 0.0)
    y2 = jnp.where(ok, y2_ref[...], 0.0)

    off = labels.astype(jnp.float32) * _OFF
    ox1 = x1 + off
    oy1 = y1 + off
    ox2 = x2 + off
    oy2 = y2 + off
    area = jnp.maximum(ox2 - ox1, 0.0) * jnp.maximum(oy2 - oy1, 0.0)

    lane = jax.lax.broadcasted_iota(jnp.int32, (1, 1, _L), 2)
    zrow = jnp.zeros((_B, 1, _L), jnp.float32)

    def sel_body(i, st):
        alive_i, avail_i, ob1, ob2, ob3, ob4, osc, olb = st
        alive = alive_i > 0
        avail = avail_i > 0
        keyv = jnp.where(alive, s, jnp.where(avail, s - 4.0, -1e9))
        mk = _rmax(keyv)
        j = _rmin(jnp.where(keyv == mk, gidx, _NP))
        one = gidx == j
        is1 = mk > 0.0
        z = jnp.float32(0.0)
        gx1 = _rsum(jnp.where(one, x1, z))
        gy1 = _rsum(jnp.where(one, y1, z))
        gx2 = _rsum(jnp.where(one, x2, z))
        gy2 = _rsum(jnp.where(one, y2, z))
        glb = _rsum(jnp.where(one, labels, 0))
        gsc = _rsum(jnp.where(one, s, z))
        ja = _rsum(jnp.where(one, area, z))
        offj = glb.astype(jnp.float32) * _OFF
        ix1 = jnp.maximum(ox1, gx1 + offj)
        iy1 = jnp.maximum(oy1, gy1 + offj)
        ix2 = jnp.minimum(ox2, gx2 + offj)
        iy2 = jnp.minimum(oy2, gy2 + offj)
        inter = jnp.maximum(ix2 - ix1, 0.0) * jnp.maximum(iy2 - iy1, 0.0)
        iou = inter / jnp.maximum(area + ja - inter, 1e-9)
        supp = iou > _IOU_TH
        alive_i = (alive & ~((supp & is1) | one)).astype(jnp.int32)
        avail_i = (avail & ~one).astype(jnp.int32)
        put = lane == i
        osc = jnp.where(put, jnp.where(is1, gsc, -1.0), osc)
        ob1 = jnp.where(put, gx1, ob1)
        ob2 = jnp.where(put, gy1, ob2)
        ob3 = jnp.where(put, gx2, ob3)
        ob4 = jnp.where(put, gy2, ob4)
        olb = jnp.where(put, glb, olb)
        return (alive_i, avail_i, ob1, ob2, ob3, ob4, osc, olb)

    alive0 = (ok & (s > 0.0)).astype(jnp.int32)
    st = jax.lax.fori_loop(
        0,
        _POST,
        sel_body,
        (alive0, jnp.broadcast_to(ok, (_B, _CR, _CL)).astype(jnp.int32),
         zrow, zrow, zrow, zrow, zrow,
         jnp.zeros((_B, 1, _L), jnp.int32)),
    )
    _, _, ob1, ob2, ob3, ob4, osc, olb = st
    bo_ref[:, 0:1, :] = ob1
    bo_ref[:, 1:2, :] = ob2
    bo_ref[:, 2:3, :] = ob3
    bo_ref[:, 3:4, :] = ob4
    bo_ref[:, 4:8, :] = jnp.zeros((_B, 4, _L), jnp.float32)
    so_ref[:, 0:1, :] = osc
    so_ref[:, 1:8, :] = jnp.zeros((_B, 7, _L), jnp.float32)
    lo_ref[:, 0:1, :] = olb
    lo_ref[:, 1:8, :] = jnp.zeros((_B, 7, _L), jnp.int32)


def _build_decode(interpret=False):
    return pl.pallas_call(
        _decode_kernel,
        grid=(_B, _R // _RT),
        in_specs=[
            pl.BlockSpec((1, _C, _RT, _L), lambda b, t: (b, 0, t, 0)),
            pl.BlockSpec((1, 4, _RT, _L), lambda b, t: (b, 0, t, 0)),
            pl.BlockSpec((1, 4, _RT, _L), lambda b, t: (b, 0, t, 0)),
        ],
        out_specs=[
            pl.BlockSpec((1, _RT, _L), lambda b, t: (b, t, 0)),
            pl.BlockSpec((1, _RT, _L), lambda b, t: (b, t, 0)),
            pl.BlockSpec((1, 4, _RT, _L), lambda b, t: (b, 0, t, 0)),
        ],
        out_shape=[
            jax.ShapeDtypeStruct((_B, _R, _L), jnp.float32),
            jax.ShapeDtypeStruct((_B, _R, _L), jnp.int32),
            jax.ShapeDtypeStruct((_B, 4, _R, _L), jnp.float32),
        ],
        interpret=interpret,
    )


def _build_topk(interpret=False):
    return pl.pallas_call(
        _topk_kernel,
        out_shape=[
            jax.ShapeDtypeStruct((_B, _R, _L), jnp.int32),
            jax.ShapeDtypeStruct((_B, 16, _L), jnp.int32),
        ],
        interpret=interpret,
    )


def _build_nms(interpret=False):
    return pl.pallas_call(
        _nms_kernel,
        out_shape=[
            jax.ShapeDtypeStruct((_B, 8, _L), jnp.float32),
            jax.ShapeDtypeStruct((_B, 8, _L), jnp.float32),
            jax.ShapeDtypeStruct((_B, 8, _L), jnp.int32),
        ],
        interpret=interpret,
    )


def _build_compact():
    f32 = jnp.float32
    i32 = jnp.int32
    n = _B * _CP
    return pl.kernel(
        _compact_kernel,
        out_type=[
            jax.ShapeDtypeStruct((n,), f32),
            jax.ShapeDtypeStruct((n,), i32),
            jax.ShapeDtypeStruct((n,), i32),
            jax.ShapeDtypeStruct((n,), f32),
            jax.ShapeDtypeStruct((n,), f32),
            jax.ShapeDtypeStruct((n,), f32),
            jax.ShapeDtypeStruct((n,), f32),
        ],
        scratch_types=[
            pltpu.VMEM((16 * _L,), i32),
            pltpu.VMEM((_CHUNK,), i32),
            pltpu.VMEM((_CHUNK,), i32),
            pltpu.VMEM((_CHUNK,), i32),
            pltpu.VMEM((_CHUNK,), f32),
            pltpu.VMEM((_CHUNK,), i32),
            pltpu.SemaphoreType.DMA,
        ],
        mesh=plsc.VectorSubcoreMesh(core_axis_name="c", subcore_axis_name="s"),
        compiler_params=pltpu.CompilerParams(needs_layout_passes=False),
    )


def _prep(x):
    xt = jnp.transpose(x, (0, 2, 1))
    xt = jnp.pad(xt, ((0, 0), (0, 0), (0, _NP - _N)))
    return xt.reshape(_B, xt.shape[1], _R, _L)


def _forward(boxes, scores, regressions, interpret=False):
    s, lb, xy = _build_decode(interpret)(
        _prep(scores), _prep(boxes), _prep(regressions)
    )
    in2k, prt = _build_topk(interpret)(s)
    flat = lambda a: a.reshape(-1)
    sc_out = _build_compact()(
        flat(in2k), flat(prt), flat(s), flat(lb),
        flat(xy[:, 0]), flat(xy[:, 1]), flat(xy[:, 2]), flat(xy[:, 3]),
    )
    cs, cg, clb, cx1, cy1, cx2, cy2 = [
        a.reshape(_B, _CP)[:, :_CR * _CL].reshape(_B, _CR, _CL) for a in sc_out
    ]
    bo, so, lo = _build_nms(interpret)(cs, cg, clb, cx1, cy1, cx2, cy2)
    pred_boxes = jnp.transpose(bo[:, :4, :_POST], (0, 2, 1))
    return pred_boxes, so[:, 0, :_POST], lo[:, 0, :_POST]


@jax.jit
def _run(boxes, scores, regressions):
    return _forward(boxes, scores, regressions)


def kernel(boxes, scores, regressions):
    return _run(boxes, scores, regressions)


# speedup vs baseline: 1.3695x; 1.3695x over previous
import jax
import jax.numpy as jnp
from jax.experimental import pallas as pl
from jax.experimental.pallas import tpu as pltpu
from jax.experimental.pallas import tpu_sc as plsc

_B, _N, _C = 4, 20000, 80
_R, _L = 160, 128
_RT = 16
_NP = _R * _L
_SCORE_TH = 0.05
_IOU_TH = 0.5
_PRE = 2000
_POST = 100
_IMG = 512.0
_OFF = 2.0 * _IMG

_CR, _CL = 16, 128
_CP = 2560
_TRASH = 2400
_CHUNK = _NP // 16


def _decode_kernel(sc_ref, bx_ref, rg_ref, s_ref, lb_ref, xy_ref):
    def cls_body(c, carry):
        best, besti = carry
        v = sc_ref[0, c]
        gt = v > best
        return jnp.where(gt, v, best), jnp.where(gt, c, besti)

    best0 = jnp.full((_RT, _L), -jnp.inf, jnp.float32)
    besti0 = jnp.zeros((_RT, _L), jnp.int32)
    best, labels = jax.lax.fori_loop(0, _C, cls_body, (best0, besti0))

    t = pl.program_id(1)
    rowi = jax.lax.broadcasted_iota(jnp.int32, (_RT, _L), 0) + t * _RT
    coli = jax.lax.broadcasted_iota(jnp.int32, (_RT, _L), 1)
    gidx = rowi * _L + coli
    pad = gidx >= _N
    valid = best > _SCORE_TH
    s_ref[0] = jnp.where(pad, -2.0, jnp.where(valid, best, -1.0))
    lb_ref[0] = labels

    bx1 = bx_ref[0, 0]
    by1 = bx_ref[0, 1]
    bx2 = bx_ref[0, 2]
    by2 = bx_ref[0, 3]
    dx = rg_ref[0, 0] * 0.1
    dy = rg_ref[0, 1] * 0.1
    dw = rg_ref[0, 2] * 0.2
    dh = rg_ref[0, 3] * 0.2
    w = bx2 - bx1
    h = by2 - by1
    cx = bx1 + 0.5 * w
    cy = by1 + 0.5 * h
    pcx = cx + dx * w
    pcy = cy + dy * h
    pw = w * jnp.exp(dw)
    ph = h * jnp.exp(dh)
    xy_ref[0, 0] = jnp.clip(pcx - 0.5 * pw, 0.0, _IMG)
    xy_ref[0, 1] = jnp.clip(pcy - 0.5 * ph, 0.0, _IMG)
    xy_ref[0, 2] = jnp.clip(pcx + 0.5 * pw, 0.0, _IMG)
    xy_ref[0, 3] = jnp.clip(pcy + 0.5 * ph, 0.0, _IMG)


def _rsum(x):
    return jnp.sum(jnp.sum(x, axis=1, keepdims=True), axis=2, keepdims=True)


def _rmax(x):
    return jnp.max(jnp.max(x, axis=1, keepdims=True), axis=2, keepdims=True)


def _rmin(x):
    return jnp.min(jnp.min(x, axis=1, keepdims=True), axis=2, keepdims=True)


def _topk_kernel(s_ref, in2k_ref, base_ref):
    s = s_ref[...]
    rowi = jax.lax.broadcasted_iota(jnp.int32, (1, _R, _L), 1)
    coli = jax.lax.broadcasted_iota(jnp.int32, (1, _R, _L), 2)
    gidx = rowi * _L + coli

    bits = jax.lax.bitcast_convert_type(s, jnp.int32)
    key = jnp.where(bits >= 0, bits, -1 - (bits & jnp.int32(0x7FFFFFFF)))

    def bs_body(_, lohi):
        lo, hi = lohi
        mid = (lo >> 1) + (hi >> 1) + (lo & hi & 1)
        big = _rsum((key > mid).astype(jnp.int32)) >= _PRE
        return jnp.where(big, mid, lo), jnp.where(big, hi, mid)

    lo0 = jnp.full((_B, 1, 1), -(2**31), jnp.int32)
    hi0 = jnp.full((_B, 1, 1), 2**31 - 1, jnp.int32)
    _, tau = jax.lax.fori_loop(0, 32, bs_body, (lo0, hi0))
    n1 = _rsum((key > tau).astype(jnp.int32))
    extra = _PRE - n1
    eq = key == tau

    def bs2_body(_, lohi):
        lo, hi = lohi
        mid = (lo + hi) >> 1
        geq = _rsum((eq & (gidx < mid)).astype(jnp.int32)) >= extra
        return jnp.where(geq, lo, mid), jnp.where(geq, mid, hi)

    _, mstar = jax.lax.fori_loop(
        0, 16, bs2_body,
        (jnp.zeros((_B, 1, 1), jnp.int32), jnp.full((_B, 1, 1), _NP, jnp.int32)),
    )
    in2k = ((key > tau) | (eq & (gidx < mstar))).astype(jnp.int32)
    in2k_ref[...] = in2k
    acc = jnp.zeros((_B, 1, 1), jnp.int32)
    for t in range(16):
        base_ref[:, t:t + 1, :] = jnp.broadcast_to(acc, (_B, 1, _L))
        acc = acc + _rsum(in2k[:, t * 10:(t + 1) * 10, :])


def _compact_kernel(mask_hbm, base_hbm, gi_hbm, s_hbm, lb_hbm, x1_hbm,
                    y1_hbm, x2_hbm, y2_hbm,
                    so_hbm, go_hbm, lo_hbm, xo1_hbm, yo1_hbm, xo2_hbm,
                    yo2_hbm,
                    b16_v, mask_v, pos_v, gi_v, s_v, lb_v, x1_v, y1_v,
                    x2_v, y2_v, lsem, ssem):
    c = jax.lax.axis_index("c")
    t = jax.lax.axis_index("s")
    off_t = t * _CHUNK
    pltpu.sync_copy(gi_hbm.at[pl.ds(off_t, _CHUNK)], gi_v)

    for b_loc in range(2):
        b = c * 2 + b_loc
        off_in = b * _NP + off_t
        loads = [
            pltpu.async_copy(
                base_hbm.at[pl.ds((b * 16 + t) * _L, 16)], b16_v, lsem),
            pltpu.async_copy(mask_hbm.at[pl.ds(off_in, _CHUNK)], mask_v,
                             lsem),
            pltpu.async_copy(s_hbm.at[pl.ds(off_in, _CHUNK)], s_v, lsem),
            pltpu.async_copy(lb_hbm.at[pl.ds(off_in, _CHUNK)], lb_v, lsem),
            pltpu.async_copy(x1_hbm.at[pl.ds(off_in, _CHUNK)], x1_v, lsem),
            pltpu.async_copy(y1_hbm.at[pl.ds(off_in, _CHUNK)], y1_v, lsem),
            pltpu.async_copy(x2_hbm.at[pl.ds(off_in, _CHUNK)], x2_v, lsem),
            pltpu.async_copy(y2_hbm.at[pl.ds(off_in, _CHUNK)], y2_v, lsem),
        ]
        for h in loads:
            h.wait()

        run = b16_v[pl.ds(0, 16)][0]
        for k in range(_CHUNK // 16):
            m = mask_v[pl.ds(k * 16, 16)]
            incl = plsc.cumsum(m)
            pos = (incl - m) + run
            pos_v[pl.ds(k * 16, 16)] = jnp.where(
                m > 0, b * _CP + pos, b * _CP + _TRASH)
            run = run + jnp.sum(m)

        stores = [
            pltpu.async_copy(s_v, so_hbm.at[pos_v], ssem),
            pltpu.async_copy(gi_v, go_hbm.at[pos_v], ssem),
            pltpu.async_copy(lb_v, lo_hbm.at[pos_v], ssem),
            pltpu.async_copy(x1_v, xo1_hbm.at[pos_v], ssem),
            pltpu.async_copy(y1_v, yo1_hbm.at[pos_v], ssem),
            pltpu.async_copy(x2_v, xo2_hbm.at[pos_v], ssem),
            pltpu.async_copy(y2_v, yo2_hbm.at[pos_v], ssem),
        ]
        for h in stores:
            h.wait()


def _nms_kernel(s_ref, g_ref, lb_ref, x1_ref, y1_ref, x2_ref, y2_ref,
                bo_ref, so_ref, lo_ref):
    rowi = jax.lax.broadcasted_iota(jnp.int32, (1, _CR, _CL), 1)
    coli = jax.lax.broadcasted_iota(jnp.int32, (1, _CR, _CL), 2)
    cpos = rowi * _CL + coli
    ok = cpos < _PRE
    s = jnp.where(ok, s_ref[...], -2.0)
    gidx = jnp.where(ok, g_ref[...], _NP)
    labels = jnp.where(ok, lb_ref[...], 0)
    x1 = jnp.where(ok, x1_ref[...], 0.0)
    y1 = jnp.where(ok, y1_ref[...], 0.0)
    x2 = jnp.where(ok, x2_ref[...], 0.0)
    y2 = jnp.where(ok, y2_ref[...], 0.0)

    off = labels.astype(jnp.float32) * _OFF
    ox1 = x1 + off
    oy1 = y1 + off
    ox2 = x2 + off
    oy2 = y2 + off
    area = jnp.maximum(ox2 - ox1, 0.0) * jnp.maximum(oy2 - oy1, 0.0)

    lane = jax.lax.broadcasted_iota(jnp.int32, (1, 1, _L), 2)
    zrow = jnp.zeros((_B, 1, _L), jnp.float32)

    def sel_body(i, st):
        alive_i, avail_i, ob1, ob2, ob3, ob4, osc, olb = st
        alive = alive_i > 0
        avail = avail_i > 0
        keyv = jnp.where(alive, s, jnp.where(avail, s - 4.0, -1e9))
        mk = _rmax(keyv)
        j = _rmin(jnp.where(keyv == mk, gidx, _NP))
        one = gidx == j
        is1 = mk > 0.0
        z = jnp.float32(0.0)
        gx1 = _rsum(jnp.where(one, x1, z))
        gy1 = _rsum(jnp.where(one, y1, z))
        gx2 = _rsum(jnp.where(one, x2, z))
        gy2 = _rsum(jnp.where(one, y2, z))
        glb = _rsum(jnp.where(one, labels, 0))
        gsc = _rsum(jnp.where(one, s, z))
        ja = _rsum(jnp.where(one, area, z))
        offj = glb.astype(jnp.float32) * _OFF
        ix1 = jnp.maximum(ox1, gx1 + offj)
        iy1 = jnp.maximum(oy1, gy1 + offj)
        ix2 = jnp.minimum(ox2, gx2 + offj)
        iy2 = jnp.minimum(oy2, gy2 + offj)
        inter = jnp.maximum(ix2 - ix1, 0.0) * jnp.maximum(iy2 - iy1, 0.0)
        iou = inter / jnp.maximum(area + ja - inter, 1e-9)
        supp = iou > _IOU_TH
        alive_i = (alive & ~((supp & is1) | one)).astype(jnp.int32)
        avail_i = (avail & ~one).astype(jnp.int32)
        put = lane == i
        osc = jnp.where(put, jnp.where(is1, gsc, -1.0), osc)
        ob1 = jnp.where(put, gx1, ob1)
        ob2 = jnp.where(put, gy1, ob2)
        ob3 = jnp.where(put, gx2, ob3)
        ob4 = jnp.where(put, gy2, ob4)
        olb = jnp.where(put, glb, olb)
        return (alive_i, avail_i, ob1, ob2, ob3, ob4, osc, olb)

    alive0 = (ok & (s > 0.0)).astype(jnp.int32)
    st = jax.lax.fori_loop(
        0,
        _POST,
        sel_body,
        (alive0, jnp.broadcast_to(ok, (_B, _CR, _CL)).astype(jnp.int32),
         zrow, zrow, zrow, zrow, zrow,
         jnp.zeros((_B, 1, _L), jnp.int32)),
    )
    _, _, ob1, ob2, ob3, ob4, osc, olb = st
    bo_ref[:, 0:1, :] = ob1
    bo_ref[:, 1:2, :] = ob2
    bo_ref[:, 2:3, :] = ob3
    bo_ref[:, 3:4, :] = ob4
    bo_ref[:, 4:8, :] = jnp.zeros((_B, 4, _L), jnp.float32)
    so_ref[:, 0:1, :] = osc
    so_ref[:, 1:8, :] = jnp.zeros((_B, 7, _L), jnp.float32)
    lo_ref[:, 0:1, :] = olb
    lo_ref[:, 1:8, :] = jnp.zeros((_B, 7, _L), jnp.int32)


def _build_decode(interpret=False):
    return pl.pallas_call(
        _decode_kernel,
        grid=(_B, _R // _RT),
        in_specs=[
            pl.BlockSpec((1, _C, _RT, _L), lambda b, t: (b, 0, t, 0)),
            pl.BlockSpec((1, 4, _RT, _L), lambda b, t: (b, 0, t, 0)),
            pl.BlockSpec((1, 4, _RT, _L), lambda b, t: (b, 0, t, 0)),
        ],
        out_specs=[
            pl.BlockSpec((1, _RT, _L), lambda b, t: (b, t, 0)),
            pl.BlockSpec((1, _RT, _L), lambda b, t: (b, t, 0)),
            pl.BlockSpec((1, 4, _RT, _L), lambda b, t: (b, 0, t, 0)),
        ],
        out_shape=[
            jax.ShapeDtypeStruct((_B, _R, _L), jnp.float32),
            jax.ShapeDtypeStruct((_B, _R, _L), jnp.int32),
            jax.ShapeDtypeStruct((_B, 4, _R, _L), jnp.float32),
        ],
        interpret=interpret,
    )


def _build_topk(interpret=False):
    return pl.pallas_call(
        _topk_kernel,
        out_shape=[
            jax.ShapeDtypeStruct((_B, _R, _L), jnp.int32),
            jax.ShapeDtypeStruct((_B, 16, _L), jnp.int32),
        ],
        interpret=interpret,
    )


def _build_nms(interpret=False):
    return pl.pallas_call(
        _nms_kernel,
        out_shape=[
            jax.ShapeDtypeStruct((_B, 8, _L), jnp.float32),
            jax.ShapeDtypeStruct((_B, 8, _L), jnp.float32),
            jax.ShapeDtypeStruct((_B, 8, _L), jnp.int32),
        ],
        interpret=interpret,
    )


def _build_compact():
    f32 = jnp.float32
    i32 = jnp.int32
    n = _B * _CP
    return pl.kernel(
        _compact_kernel,
        out_type=[
            jax.ShapeDtypeStruct((n,), f32),
            jax.ShapeDtypeStruct((n,), i32),
            jax.ShapeDtypeStruct((n,), i32),
            jax.ShapeDtypeStruct((n,), f32),
            jax.ShapeDtypeStruct((n,), f32),
            jax.ShapeDtypeStruct((n,), f32),
            jax.ShapeDtypeStruct((n,), f32),
        ],
        scratch_types=[
            pltpu.VMEM((16,), i32),
            pltpu.VMEM((_CHUNK,), i32),
            pltpu.VMEM((_CHUNK,), i32),
            pltpu.VMEM((_CHUNK,), i32),
            pltpu.VMEM((_CHUNK,), f32),
            pltpu.VMEM((_CHUNK,), i32),
            pltpu.VMEM((_CHUNK,), f32),
            pltpu.VMEM((_CHUNK,), f32),
            pltpu.VMEM((_CHUNK,), f32),
            pltpu.VMEM((_CHUNK,), f32),
            pltpu.SemaphoreType.DMA,
            pltpu.SemaphoreType.DMA,
        ],
        mesh=plsc.VectorSubcoreMesh(core_axis_name="c", subcore_axis_name="s"),
        compiler_params=pltpu.CompilerParams(needs_layout_passes=False),
    )


def _prep(x):
    xt = jnp.transpose(x, (0, 2, 1))
    xt = jnp.pad(xt, ((0, 0), (0, 0), (0, _NP - _N)))
    return xt.reshape(_B, xt.shape[1], _R, _L)


def _forward(boxes, scores, regressions, interpret=False):
    s, lb, xy = _build_decode(interpret)(
        _prep(scores), _prep(boxes), _prep(regressions)
    )
    in2k, bases = _build_topk(interpret)(s)
    flat = lambda a: a.reshape(-1)
    gi = jnp.arange(_NP, dtype=jnp.int32)
    sc_out = _build_compact()(
        flat(in2k), flat(bases), gi, flat(s), flat(lb),
        flat(xy[:, 0]), flat(xy[:, 1]), flat(xy[:, 2]), flat(xy[:, 3]),
    )
    cs, cg, clb, cx1, cy1, cx2, cy2 = [
        a.reshape(_B, _CP)[:, :_CR * _CL].reshape(_B, _CR, _CL) for a in sc_out
    ]
    bo, so, lo = _build_nms(interpret)(cs, cg, clb, cx1, cy1, cx2, cy2)
    pred_boxes = jnp.transpose(bo[:, :4, :_POST], (0, 2, 1))
    return pred_boxes, so[:, 0, :_POST], lo[:, 0, :_POST]


@jax.jit
def _run(boxes, scores, regressions):
    return _forward(boxes, scores, regressions)


def kernel(boxes, scores, regressions):
    return _run(boxes, scores, regressions)


# speedup vs baseline: 51.7947x; 37.8188x over previous
import jax
import jax.numpy as jnp
from jax.experimental import pallas as pl

_B, _N, _C = 4, 20000, 80
_R, _L = 160, 128
_NP = _R * _L
_SCORE_TH = 0.05
_IOU_TH = 0.5
_PRE = 2000
_POST = 100
_IMG = 512.0
_OFF = 2.0 * _IMG


def _nms_kernel(sc_ref, bx_ref, rg_ref, bo_ref, so_ref, lo_ref):

    def cls_body(c, carry):
        best, besti = carry
        v = sc_ref[0, c]
        gt = v > best
        return jnp.where(gt, v, best), jnp.where(gt, c, besti)

    best0 = jnp.full((_R, _L), -jnp.inf, jnp.float32)
    besti0 = jnp.zeros((_R, _L), jnp.int32)
    best, labels = jax.lax.fori_loop(0, _C, cls_body, (best0, besti0))

    rowi = jax.lax.broadcasted_iota(jnp.int32, (_R, _L), 0)
    coli = jax.lax.broadcasted_iota(jnp.int32, (_R, _L), 1)
    gidx = rowi * _L + coli
    pad = gidx >= _N
    valid = best > _SCORE_TH
    s = jnp.where(pad, -2.0, jnp.where(valid, best, -1.0))

    bx1 = bx_ref[0, 0]
    by1 = bx_ref[0, 1]
    bx2 = bx_ref[0, 2]
    by2 = bx_ref[0, 3]
    dx = rg_ref[0, 0] * 0.1
    dy = rg_ref[0, 1] * 0.1
    dw = rg_ref[0, 2] * 0.2
    dh = rg_ref[0, 3] * 0.2
    w = bx2 - bx1
    h = by2 - by1
    cx = bx1 + 0.5 * w
    cy = by1 + 0.5 * h
    pcx = cx + dx * w
    pcy = cy + dy * h
    pw = w * jnp.exp(dw)
    ph = h * jnp.exp(dh)
    x1 = jnp.clip(pcx - 0.5 * pw, 0.0, _IMG)
    y1 = jnp.clip(pcy - 0.5 * ph, 0.0, _IMG)
    x2 = jnp.clip(pcx + 0.5 * pw, 0.0, _IMG)
    y2 = jnp.clip(pcy + 0.5 * ph, 0.0, _IMG)

    off = labels.astype(jnp.float32) * _OFF
    ox1 = x1 + off
    oy1 = y1 + off
    ox2 = x2 + off
    oy2 = y2 + off
    area = jnp.maximum(ox2 - ox1, 0.0) * jnp.maximum(oy2 - oy1, 0.0)

    bits = jax.lax.bitcast_convert_type(s, jnp.int32)
    key = jnp.where(bits >= 0, bits, -1 - (bits & jnp.int32(0x7FFFFFFF)))

    def bs_body(_, lohi):
        lo, hi = lohi
        mid = (lo >> 1) + (hi >> 1) + (lo & hi & 1)
        big = jnp.sum((key > mid).astype(jnp.int32)) >= _PRE
        return jnp.where(big, mid, lo), jnp.where(big, hi, mid)

    _, tau = jax.lax.fori_loop(
        0, 32, bs_body, (jnp.int32(-(2**31)), jnp.int32(2**31 - 1))
    )
    n1 = jnp.sum((key > tau).astype(jnp.int32))
    extra = _PRE - n1
    eq = key == tau

    def bs2_body(_, lohi):
        lo, hi = lohi
        mid = (lo + hi) >> 1
        geq = jnp.sum((eq & (gidx < mid)).astype(jnp.int32)) >= extra
        return jnp.where(geq, lo, mid), jnp.where(geq, mid, hi)

    _, mstar = jax.lax.fori_loop(
        0, 16, bs2_body, (jnp.int32(0), jnp.int32(_NP))
    )
    in2k = (key > tau) | (eq & (gidx < mstar))

    lane = jax.lax.broadcasted_iota(jnp.int32, (1, _L), 1)
    zrow = jnp.zeros((1, _L), jnp.float32)

    def sel_body(i, st):
        alive_i, avail_i, ob1, ob2, ob3, ob4, osc, olb = st
        alive = alive_i > 0
        avail = avail_i > 0
        keyv = jnp.where(alive, s, jnp.where(avail, s - 4.0, -1e9))
        mk = jnp.max(keyv)
        j = jnp.min(jnp.where(keyv == mk, gidx, _NP))
        one = gidx == j
        is1 = mk > 0.0
        z = jnp.float32(0.0)
        gx1 = jnp.sum(jnp.where(one, x1, z))
        gy1 = jnp.sum(jnp.where(one, y1, z))
        gx2 = jnp.sum(jnp.where(one, x2, z))
        gy2 = jnp.sum(jnp.where(one, y2, z))
        glb = jnp.sum(jnp.where(one, labels, 0))
        gsc = jnp.sum(jnp.where(one, s, z))
        ja = jnp.sum(jnp.where(one, area, z))
        offj = glb.astype(jnp.float32) * _OFF
        ix1 = jnp.maximum(ox1, gx1 + offj)
        iy1 = jnp.maximum(oy1, gy1 + offj)
        ix2 = jnp.minimum(ox2, gx2 + offj)
        iy2 = jnp.minimum(oy2, gy2 + offj)
        inter = jnp.maximum(ix2 - ix1, 0.0) * jnp.maximum(iy2 - iy1, 0.0)
        iou = inter / jnp.maximum(area + ja - inter, 1e-9)
        supp = iou > _IOU_TH
        alive_i = (alive & ~((supp & is1) | one)).astype(jnp.int32)
        avail_i = (avail & ~one).astype(jnp.int32)
        put = lane == i
        osc = jnp.where(put, jnp.where(is1, gsc, -1.0), osc)
        ob1 = jnp.where(put, gx1, ob1)
        ob2 = jnp.where(put, gy1, ob2)
        ob3 = jnp.where(put, gx2, ob3)
        ob4 = jnp.where(put, gy2, ob4)
        olb = jnp.where(put, glb, olb)
        return (alive_i, avail_i, ob1, ob2, ob3, ob4, osc, olb)

    alive0 = (in2k & (s > 0.0)).astype(jnp.int32)
    st = jax.lax.fori_loop(
        0,
        _POST,
        sel_body,
        (alive0, in2k.astype(jnp.int32), zrow, zrow, zrow, zrow, zrow,
         jnp.zeros((1, _L), jnp.int32)),
    )
    _, _, ob1, ob2, ob3, ob4, osc, olb = st
    bo_ref[0, 0:1, :] = ob1
    bo_ref[0, 1:2, :] = ob2
    bo_ref[0, 2:3, :] = ob3
    bo_ref[0, 3:4, :] = ob4
    bo_ref[0, 4:8, :] = jnp.zeros((4, _L), jnp.float32)
    so_ref[0, 0:1, :] = osc
    so_ref[0, 1:8, :] = jnp.zeros((7, _L), jnp.float32)
    lo_ref[0, 0:1, :] = olb
    lo_ref[0, 1:8, :] = jnp.zeros((7, _L), jnp.int32)


def _build(interpret=False):
    return pl.pallas_call(
        _nms_kernel,
        grid=(_B,),
        in_specs=[
            pl.BlockSpec((1, _C, _R, _L), lambda b: (b, 0, 0, 0)),
            pl.BlockSpec((1, 4, _R, _L), lambda b: (b, 0, 0, 0)),
            pl.BlockSpec((1, 4, _R, _L), lambda b: (b, 0, 0, 0)),
        ],
        out_specs=[
            pl.BlockSpec((1, 8, _L), lambda b: (b, 0, 0)),
            pl.BlockSpec((1, 8, _L), lambda b: (b, 0, 0)),
            pl.BlockSpec((1, 8, _L), lambda b: (b, 0, 0)),
        ],
        out_shape=[
            jax.ShapeDtypeStruct((_B, 8, _L), jnp.float32),
            jax.ShapeDtypeStruct((_B, 8, _L), jnp.float32),
            jax.ShapeDtypeStruct((_B, 8, _L), jnp.int32),
        ],
        interpret=interpret,
    )


def _prep(x):
    xt = jnp.transpose(x, (0, 2, 1))
    xt = jnp.pad(xt, ((0, 0), (0, 0), (0, _NP - _N)))
    return xt.reshape(_B, xt.shape[1], _R, _L)


@jax.jit
def _run(boxes, scores, regressions):
    bo, so, lo = _build()(_prep(scores), _prep(boxes), _prep(regressions))
    pred_boxes = jnp.transpose(bo[:, :4, :_POST], (0, 2, 1))
    return pred_boxes, so[:, 0, :_POST], lo[:, 0, :_POST]


def kernel(boxes, scores, regressions):
    return _run(boxes, scores, regressions)
